# Initial kernel scaffold; baseline (speedup 1.0000x reference)
import functools

import jax
import jax.numpy as jnp
from jax import lax
from jax.experimental import pallas as pl
from jax.experimental.pallas import tpu as pltpu
from jax.experimental.pallas import tpu_sc as plsc

L_IN = 100000
D = 128
S = 1024 * 200
NC, NS = 2, 16
NW = NC * NS
PER_W = S // NW
CHUNK = 128
NCH = PER_W // CHUNK
LANES = 16


def _sc_body(t_hbm, tab_hbm, out_hbm, t_v, w_v, idx0_v, idx1_v,
             rows0_v, rows1_v, sem):
    wid = lax.axis_index("s") * NC + lax.axis_index("c")
    base = wid * PER_W
    pltpu.sync_copy(t_hbm.at[pl.ds(base, PER_W)], t_v)

    def precompute(c, carry):
        for j in range(CHUNK // LANES):
            off = c * CHUNK + j * LANES
            tv = t_v[pl.ds(off, LANES)]
            g = tv * 2.0 - 1.0
            x = ((g + 1.0) * float(L_IN) - 1.0) * 0.5
            xc = jnp.minimum(jnp.maximum(x, 0.0), float(L_IN - 1))
            i0 = xc.astype(jnp.int32)
            wv = xc - i0.astype(jnp.float32)
            i1 = jnp.minimum(i0 + 1, L_IN - 1)
            idx0_v[c, pl.ds(j * LANES, LANES)] = i0
            idx1_v[c, pl.ds(j * LANES, LANES)] = i1
            w_v[pl.ds(off, LANES)] = wv
        return carry

    lax.fori_loop(0, NCH, precompute, 0)

    def chunk_body(c, carry):
        cp0 = pltpu.async_copy(tab_hbm.at[idx0_v.at[c]], rows0_v, sem)
        cp1 = pltpu.async_copy(tab_hbm.at[idx1_v.at[c]], rows1_v, sem)
        cp0.wait()
        cp1.wait()

        def s_body(s, inner):
            widx = jnp.full((LANES,), c * CHUNK + s, jnp.int32)
            wv = plsc.load_gather(w_v, [widx])
            for col in range(D // LANES):
                r0 = rows0_v[s, pl.ds(col * LANES, LANES)]
                r1 = rows1_v[s, pl.ds(col * LANES, LANES)]
                rows0_v[s, pl.ds(col * LANES, LANES)] = r0 + wv * (r1 - r0)
            return inner

        lax.fori_loop(0, CHUNK, s_body, 0)
        pltpu.sync_copy(rows0_v, out_hbm.at[pl.ds(base + c * CHUNK, CHUNK)])
        return carry

    lax.fori_loop(0, NCH, chunk_body, 0)


@jax.jit
def _sc_embed(t_flat, latent_codes):
    mesh = plsc.VectorSubcoreMesh(core_axis_name="c", subcore_axis_name="s")
    fn = pl.kernel(
        _sc_body,
        out_type=jax.ShapeDtypeStruct((S, D), jnp.float32),
        mesh=mesh,
        scratch_types=[
            pltpu.VMEM((PER_W,), jnp.float32),
            pltpu.VMEM((PER_W,), jnp.float32),
            pltpu.VMEM((NCH, CHUNK), jnp.int32),
            pltpu.VMEM((NCH, CHUNK), jnp.int32),
            pltpu.VMEM((CHUNK, D), jnp.float32),
            pltpu.VMEM((CHUNK, D), jnp.float32),
            pltpu.SemaphoreType.DMA,
        ],
    )
    return fn(t_flat, latent_codes)


def kernel(t, latent_codes):
    out = _sc_embed(t.reshape(S), latent_codes)
    return out.reshape(t.shape[:-1] + (D,))


# speedup vs baseline: 2.8750x; 2.8750x over previous
import functools

import jax
import jax.numpy as jnp
from jax import lax
from jax.experimental import pallas as pl
from jax.experimental.pallas import tpu as pltpu
from jax.experimental.pallas import tpu_sc as plsc

L_IN = 100000
D = 128
S = 1024 * 200
NC, NS = 2, 16
NW = NC * NS
PER_W = S // NW
CHUNK = 128
NCH = PER_W // CHUNK
LANES = 16


def _sc_body(t_hbm, tab_hbm, out_hbm, t_v, w_v, idx0_v, idx1_v,
             rows0_v, rows1_v, sem):
    wid = lax.axis_index("s") * NC + lax.axis_index("c")
    base = wid * PER_W
    pltpu.sync_copy(t_hbm.at[pl.ds(base, PER_W)], t_v)

    def precompute(c, carry):
        for j in range(CHUNK // LANES):
            off = c * CHUNK + j * LANES
            tv = t_v[pl.ds(off, LANES)]
            g = tv * 2.0 - 1.0
            x = ((g + 1.0) * float(L_IN) - 1.0) * 0.5
            xc = jnp.minimum(jnp.maximum(x, 0.0), float(L_IN - 1))
            i0 = xc.astype(jnp.int32)
            wv = xc - i0.astype(jnp.float32)
            i1 = jnp.minimum(i0 + 1, L_IN - 1)
            idx0_v[c, pl.ds(j * LANES, LANES)] = i0
            idx1_v[c, pl.ds(j * LANES, LANES)] = i1
            w_v[pl.ds(off, LANES)] = wv
        return carry

    lax.fori_loop(0, NCH, precompute, 0)

    def chunk_body(c, carry):
        cp0 = pltpu.async_copy(tab_hbm.at[idx0_v.at[c]], rows0_v, sem)
        cp1 = pltpu.async_copy(tab_hbm.at[idx1_v.at[c]], rows1_v, sem)
        cp0.wait()
        cp1.wait()

        def grp_body(j, inner):
            wv16 = w_v[pl.ds(c * CHUNK + j * LANES, LANES)]
            for k in range(LANES):
                s = j * LANES + k
                wv = jnp.full((LANES,), wv16[k], jnp.float32)
                for col in range(D // LANES):
                    r0 = rows0_v[s, pl.ds(col * LANES, LANES)]
                    r1 = rows1_v[s, pl.ds(col * LANES, LANES)]
                    rows0_v[s, pl.ds(col * LANES, LANES)] = r0 + wv * (r1 - r0)
            return inner

        lax.fori_loop(0, CHUNK // LANES, grp_body, 0)
        pltpu.sync_copy(rows0_v, out_hbm.at[pl.ds(base + c * CHUNK, CHUNK)])
        return carry

    lax.fori_loop(0, NCH, chunk_body, 0)


@jax.jit
def _sc_embed(t_flat, latent_codes):
    mesh = plsc.VectorSubcoreMesh(core_axis_name="c", subcore_axis_name="s")
    fn = pl.kernel(
        _sc_body,
        out_type=jax.ShapeDtypeStruct((S, D), jnp.float32),
        mesh=mesh,
        scratch_types=[
            pltpu.VMEM((PER_W,), jnp.float32),
            pltpu.VMEM((PER_W,), jnp.float32),
            pltpu.VMEM((NCH, CHUNK), jnp.int32),
            pltpu.VMEM((NCH, CHUNK), jnp.int32),
            pltpu.VMEM((CHUNK, D), jnp.float32),
            pltpu.VMEM((CHUNK, D), jnp.float32),
            pltpu.SemaphoreType.DMA,
        ],
    )
    return fn(t_flat, latent_codes)


def kernel(t, latent_codes):
    out = _sc_embed(t.reshape(S), latent_codes)
    return out.reshape(t.shape[:-1] + (D,))


# speedup vs baseline: 11.0135x; 3.8308x over previous
import jax
import jax.numpy as jnp
from jax import lax
from jax.experimental import pallas as pl
from jax.experimental.pallas import tpu as pltpu
from jax.experimental.pallas import tpu_sc as plsc

L_IN = 100000
D = 128
S = 1024 * 200
NC, NS = 2, 16
NW = NC * NS
PER_W = S // NW
CHUNK = 128
NCH = PER_W // CHUNK
LANES = 16
NBUF = 2


def _sc_body(t_hbm, tab_hbm, out_hbm, w_v, idx0_v, idx1_v,
             rows0, rows1, outb, gsem, osem):
    wid = lax.axis_index("s") * NC + lax.axis_index("c")
    base = wid * PER_W

    pltpu.sync_copy(t_hbm.at[pl.ds(base, PER_W)], w_v.at[pl.ds(0, PER_W)])

    def precompute(c, carry):
        for j in range(CHUNK // LANES):
            off = c * CHUNK + j * LANES
            tv = w_v[pl.ds(off, LANES)]
            g = tv * 2.0 - 1.0
            x = ((g + 1.0) * float(L_IN) - 1.0) * 0.5
            xc = jnp.minimum(jnp.maximum(x, 0.0), float(L_IN - 1))
            i0 = xc.astype(jnp.int32)
            wv = xc - i0.astype(jnp.float32)
            i1 = jnp.minimum(i0 + 1, L_IN - 1)
            idx0_v[c, pl.ds(j * LANES, LANES)] = i0
            idx1_v[c, pl.ds(j * LANES, LANES)] = i1
            w_v[pl.ds(off, LANES)] = wv
        return carry

    lax.fori_loop(0, NCH, precompute, 0)

    def issue_gathers(c, b):
        pltpu.async_copy(tab_hbm.at[idx0_v.at[c]], rows0[b], gsem[b])
        pltpu.async_copy(tab_hbm.at[idx1_v.at[c]], rows1[b], gsem[b])

    def wait_gathers(c, b):
        pltpu.make_async_copy(tab_hbm.at[idx0_v.at[c]], rows0[b], gsem[b]).wait()
        pltpu.make_async_copy(tab_hbm.at[idx1_v.at[c]], rows1[b], gsem[b]).wait()

    def issue_scatter(c, b):
        pltpu.async_copy(outb[b], out_hbm.at[pl.ds(base + c * CHUNK, CHUNK)],
                         osem[b])

    def wait_scatter(c, b):
        pltpu.make_async_copy(outb[b],
                              out_hbm.at[pl.ds(base + c * CHUNK, CHUNK)],
                              osem[b]).wait()

    def lerp(c, b):
        def s_body(s, inner):
            wv16 = w_v[pl.ds(c * CHUNK + s, LANES)]
            wb = jnp.full((LANES,), wv16[0], jnp.float32)
            for col in range(D // LANES):
                r0 = rows0[b][s, pl.ds(col * LANES, LANES)]
                r1 = rows1[b][s, pl.ds(col * LANES, LANES)]
                outb[b][s, pl.ds(col * LANES, LANES)] = r0 + wb * (r1 - r0)
            return inner

        lax.fori_loop(0, CHUNK, s_body, 0)

    for b in range(NBUF):
        issue_gathers(b, b)
    for b in range(NBUF):
        wait_gathers(b, b)
        lerp(b, b)
        issue_scatter(b, b)
        issue_gathers(b + NBUF, b)

    def main_body(m, carry):
        for b in range(NBUF):
            c = NBUF + NBUF * m + b
            wait_gathers(c, b)
            wait_scatter(c - NBUF, b)
            lerp(c, b)
            issue_scatter(c, b)
            issue_gathers(c + NBUF, b)
        return carry

    lax.fori_loop(0, (NCH - 2 * NBUF) // NBUF, main_body, 0)

    for b in range(NBUF):
        c = NCH - NBUF + b
        wait_gathers(c, b)
        wait_scatter(c - NBUF, b)
        lerp(c, b)
        issue_scatter(c, b)
    for b in range(NBUF):
        wait_scatter(NCH - NBUF + b, b)


def _body_wrap(t_hbm, tab_hbm, out_hbm, w_v, idx0_v, idx1_v,
               r0a, r0b, r1a, r1b, oa, ob, gsa, gsb, osa, osb):
    _sc_body(t_hbm, tab_hbm, out_hbm, w_v, idx0_v, idx1_v,
             (r0a, r0b), (r1a, r1b), (oa, ob), (gsa, gsb), (osa, osb))


@jax.jit
def _sc_embed(t_flat, latent_codes):
    mesh = plsc.VectorSubcoreMesh(core_axis_name="c", subcore_axis_name="s")
    fn = pl.kernel(
        _body_wrap,
        out_type=jax.ShapeDtypeStruct((S, D), jnp.float32),
        mesh=mesh,
        scratch_types=[
            pltpu.VMEM((PER_W + LANES,), jnp.float32),
            pltpu.VMEM((NCH, CHUNK), jnp.int32),
            pltpu.VMEM((NCH, CHUNK), jnp.int32),
            pltpu.VMEM((CHUNK, D), jnp.float32),
            pltpu.VMEM((CHUNK, D), jnp.float32),
            pltpu.VMEM((CHUNK, D), jnp.float32),
            pltpu.VMEM((CHUNK, D), jnp.float32),
            pltpu.VMEM((CHUNK, D), jnp.float32),
            pltpu.VMEM((CHUNK, D), jnp.float32),
            pltpu.SemaphoreType.DMA,
            pltpu.SemaphoreType.DMA,
            pltpu.SemaphoreType.DMA,
            pltpu.SemaphoreType.DMA,
        ],
    )
    return fn(t_flat, latent_codes)


def kernel(t, latent_codes):
    out = _sc_embed(t.reshape(S), latent_codes)
    return out.reshape(t.shape[:-1] + (D,))
